# Initial kernel scaffold; baseline (speedup 1.0000x reference)
import jax
import jax.numpy as jnp
from jax.experimental import pallas as pl


def kernel(features, edge_index, W_self1, W_neigh1, b1, W_self2, W_neigh2, b2):
    raise NotImplementedError("write your pallas kernel here")


# speedup vs baseline: 6.3407x; 6.3407x over previous
import functools

import jax
import jax.numpy as jnp
from jax import lax
from jax.experimental import pallas as pl
from jax.experimental.pallas import tpu as pltpu
from jax.experimental.pallas import tpu_sc as plsc

N_NODES = 10000
N_EDGES = 320000
D = 128
HID = 256
DEG_W = 16

NC = 2
NS = 16
NW = NC * NS
E_PER_W = N_EDGES // NW
CHUNK = 80
N_CHUNKS = E_PER_W // CHUNK
ROWS_PER_TILE = 624
TAIL_ROW0 = NS * ROWS_PER_TILE
TAIL_ROWS = N_NODES - TAIL_ROW0
ZROWS = 16
BLK = 1024
NBLK = -(-N_NODES // BLK)
DEG_PAD = NBLK * BLK

_MESH = plsc.VectorSubcoreMesh(
    core_axis_name="c", subcore_axis_name="s", num_cores=NC, num_subcores=NS
)


def _fill_rows(ref, nrows, ncols, value):
  vec = jnp.full((16,), value, jnp.float32)

  def body(r, _):
    for k in range(ncols // 16):
      ref[r, pl.ds(k * 16, 16)] = vec
    return 0

  lax.fori_loop(0, nrows, body, 0)


def _make_agg(with_deg):
  def body(x_hbm, src_hbm, dst_hbm, *rest):
    if with_deg:
      (out_hbm, deg_hbm, accum, src_v, dst_v, rows_v, deg_local, zero_v,
       sem) = rest
    else:
      (out_hbm, accum, src_v, dst_v, rows_v, zero_v, sem) = rest
    c = lax.axis_index("c")
    s = lax.axis_index("s")
    wid = (c * NS + s).astype(jnp.int32)
    row0 = s * ROWS_PER_TILE

    _fill_rows(zero_v, ZROWS, D, 0.0)
    if with_deg:
      zvec = jnp.zeros((16,), jnp.float32)

      def zdl(i, _):
        deg_local[0, pl.ds(i * 16, 16)] = zvec
        return 0

      lax.fori_loop(0, DEG_PAD // 16, zdl, 0)

    def zloop(i, _):
      pltpu.sync_copy(zero_v, accum.at[pl.ds(row0 + i * ZROWS, ZROWS)])
      return 0

    lax.fori_loop(0, ROWS_PER_TILE // ZROWS, zloop, 0)

    @pl.when(s == NS - 1)
    def _zero_tail():
      pltpu.sync_copy(zero_v, accum.at[pl.ds(TAIL_ROW0, TAIL_ROWS)])

    plsc.subcore_barrier()

    e0 = wid * E_PER_W

    def chunk_body(i, _):
      base = e0 + i * CHUNK
      pltpu.sync_copy(src_hbm.at[pl.ds(base, CHUNK)], src_v)
      pltpu.sync_copy(dst_hbm.at[pl.ds(base, CHUNK)], dst_v)
      pltpu.async_copy(x_hbm.at[src_v], rows_v, sem).wait()
      pltpu.sync_copy(rows_v, accum.at[dst_v], add=True)
      if with_deg:
        one16 = jnp.ones((16,), jnp.float32)
        zrow = jnp.zeros((16,), jnp.int32)
        for kk in range(CHUNK // 16):
          dvec = dst_v[pl.ds(kk * 16, 16)]
          plsc.addupdate_scatter(deg_local, [zrow, dvec], one16)
      return 0

    lax.fori_loop(0, N_CHUNKS, chunk_body, 0)
    plsc.subcore_barrier()

    pltpu.sync_copy(
        accum.at[pl.ds(row0, ROWS_PER_TILE)],
        out_hbm.at[c, pl.ds(row0, ROWS_PER_TILE)],
    )
    if with_deg:
      pltpu.sync_copy(deg_local, deg_hbm.at[wid])

    @pl.when(s == NS - 1)
    def _copy_tail():
      pltpu.sync_copy(
          accum.at[pl.ds(TAIL_ROW0, TAIL_ROWS)],
          out_hbm.at[c, pl.ds(TAIL_ROW0, TAIL_ROWS)],
      )

  out_type = [jax.ShapeDtypeStruct((NC, N_NODES, D), jnp.float32)]
  scratch = [pltpu.VMEM_SHARED((N_NODES, D), jnp.float32)]
  if with_deg:
    out_type.append(jax.ShapeDtypeStruct((NW, 1, DEG_PAD), jnp.float32))
  scratch += [
      pltpu.VMEM((CHUNK,), jnp.int32),
      pltpu.VMEM((CHUNK,), jnp.int32),
      pltpu.VMEM((CHUNK, D), jnp.float32),
  ]
  if with_deg:
    scratch.append(pltpu.VMEM((1, DEG_PAD), jnp.float32))
  scratch.append(pltpu.VMEM((ZROWS, D), jnp.float32))
  scratch.append(pltpu.SemaphoreType.DMA)

  return pl.kernel(
      body,
      out_type=tuple(out_type),
      mesh=_MESH,
      scratch_types=tuple(scratch),
      name="sage_agg_deg" if with_deg else "sage_agg",
      compiler_params=pltpu.CompilerParams(needs_layout_passes=False),
  )


_agg_with_deg = _make_agg(True)
_agg_plain = _make_agg(False)


def _mlp_body(x_ref, s1a_ref, s1b_ref, degp_ref, ws1_ref, wn1_ref,
              b1_ref, ws2_ref, wn2_ref, b2_ref, z2_ref, self2_ref, inv_ref):
  deg = jnp.sum(degp_ref[:, 0, :], axis=0)[:, None]
  inv = 1.0 / jnp.maximum(deg, 1.0)
  hn = (s1a_ref[...] + s1b_ref[...]) * inv
  h = x_ref[...] @ ws1_ref[...] + hn @ wn1_ref[...] + b1_ref[...]
  h = jnp.maximum(h, 0.0)
  z2_ref[...] = h @ wn2_ref[...]
  self2_ref[...] = h @ ws2_ref[...] + b2_ref[...]
  inv_ref[...] = jnp.broadcast_to(inv, (BLK, DEG_W))


def _row_spec(cols):
  return pl.BlockSpec((BLK, cols), lambda i: (i, 0))


def _full_spec(r, c):
  return pl.BlockSpec((r, c), lambda i: (0, 0))


_mlp = pl.pallas_call(
    _mlp_body,
    grid=(NBLK,),
    in_specs=[
        _row_spec(D), _row_spec(D), _row_spec(D),
        pl.BlockSpec((NW, 1, BLK), lambda i: (0, 0, i)),
        _full_spec(D, HID), _full_spec(D, HID), _full_spec(1, HID),
        _full_spec(HID, D), _full_spec(HID, D), _full_spec(1, D),
    ],
    out_specs=[_row_spec(D), _row_spec(D), _row_spec(DEG_W)],
    out_shape=[
        jax.ShapeDtypeStruct((N_NODES, D), jnp.float32),
        jax.ShapeDtypeStruct((N_NODES, D), jnp.float32),
        jax.ShapeDtypeStruct((N_NODES, DEG_W), jnp.float32),
    ],
)


def _fin_body(self2_ref, s2a_ref, s2b_ref, inv_ref, out_ref):
  out_ref[...] = (
      self2_ref[...]
      + (s2a_ref[...] + s2b_ref[...]) * inv_ref[...][:, :1]
  )


_fin = pl.pallas_call(
    _fin_body,
    grid=(NBLK,),
    in_specs=[_row_spec(D), _row_spec(D), _row_spec(D), _row_spec(DEG_W)],
    out_specs=_row_spec(D),
    out_shape=jax.ShapeDtypeStruct((N_NODES, D), jnp.float32),
)


def kernel(features, edge_index, W_self1, W_neigh1, b1, W_self2, W_neigh2,
           b2):
  src = edge_index[0].astype(jnp.int32)
  dst = edge_index[1].astype(jnp.int32)

  s1p, degp = _agg_with_deg(features, src, dst)
  z2, self2, inv = _mlp(
      features, s1p[0], s1p[1], degp,
      W_self1, W_neigh1, b1.reshape(1, HID),
      W_self2, W_neigh2, b2.reshape(1, D),
  )
  (s2p,) = _agg_plain(z2, src, dst)
  return _fin(self2, s2p[0], s2p[1], inv)


# speedup vs baseline: 13.1800x; 2.0786x over previous
import functools

import jax
import jax.numpy as jnp
from jax import lax
from jax.experimental import pallas as pl
from jax.experimental.pallas import tpu as pltpu
from jax.experimental.pallas import tpu_sc as plsc

N_NODES = 10000
N_EDGES = 320000
D = 128
HID = 256
DEG_W = 16

NC = 2
NS = 16
NW = NC * NS
E_PER_W = N_EDGES // NW
CHUNK = 80
STAGE_E = 2000
N_STAGES = E_PER_W // STAGE_E
SCH = STAGE_E // CHUNK
ROWS_PER_TILE = 624
TAIL_ROW0 = NS * ROWS_PER_TILE
TAIL_ROWS = N_NODES - TAIL_ROW0
ZROWS = 16
BLK = 1024
NBLK = -(-N_NODES // BLK)
DEG_PAD = NBLK * BLK

_MESH = plsc.VectorSubcoreMesh(
    core_axis_name="c", subcore_axis_name="s", num_cores=NC, num_subcores=NS
)


def _fill_rows(ref, nrows, ncols, value):
  vec = jnp.full((16,), value, jnp.float32)

  def body(r, _):
    for k in range(ncols // 16):
      ref[r, pl.ds(k * 16, 16)] = vec
    return 0

  lax.fori_loop(0, nrows, body, 0)


def _make_agg(with_deg):
  def body(x_hbm, src_hbm, dst_hbm, *rest):
    if with_deg:
      (out_hbm, deg_hbm, accum, src_stage, dst_stage, sv0, sv1, dv0, dv1,
       rows0, rows1, deg_local, zero_v, semA, semB) = rest
    else:
      (out_hbm, accum, src_stage, dst_stage, sv0, sv1, dv0, dv1,
       rows0, rows1, zero_v, semA, semB) = rest
    c = lax.axis_index("c")
    s = lax.axis_index("s")
    wid = (c * NS + s).astype(jnp.int32)
    row0 = s * ROWS_PER_TILE
    e0 = wid * E_PER_W

    _fill_rows(zero_v, ZROWS, D, 0.0)
    if with_deg:
      zvec = jnp.zeros((16,), jnp.float32)

      def zdl(i, _):
        deg_local[0, pl.ds(i * 16, 16)] = zvec
        return 0

      lax.fori_loop(0, DEG_PAD // 16, zdl, 0)

    def zloop(i, _):
      pltpu.sync_copy(zero_v, accum.at[pl.ds(row0 + i * ZROWS, ZROWS)])
      return 0

    lax.fori_loop(0, ROWS_PER_TILE // ZROWS, zloop, 0)

    @pl.when(s == NS - 1)
    def _zero_tail():
      pltpu.sync_copy(
          zero_v.at[pl.ds(0, TAIL_ROWS)], accum.at[pl.ds(TAIL_ROW0, TAIL_ROWS)]
      )

    plsc.subcore_barrier()

    def set_window(k, sv, dv):
      for t in range(CHUNK // 16):
        off = k * CHUNK + t * 16
        sv[pl.ds(t * 16, 16)] = src_stage[pl.ds(off, 16)]
        dv[pl.ds(t * 16, 16)] = dst_stage[pl.ds(off, 16)]

    def start_gather(sv, rows, sem):
      pltpu.async_copy(x_hbm.at[sv], rows, sem)

    def wait_gather(rows, sem):
      pltpu.make_async_copy(x_hbm.at[pl.ds(0, CHUNK)], rows, sem).wait()

    def consume(dv, rows, sem):
      wait_gather(rows, sem)
      pltpu.sync_copy(rows, accum.at[dv], add=True)
      if with_deg:
        one16 = jnp.ones((16,), jnp.float32)
        zrow = jnp.zeros((16,), jnp.int32)
        for kk in range(CHUNK // 16):
          dvec = dv[pl.ds(kk * 16, 16)]
          plsc.addupdate_scatter(deg_local, [zrow, dvec], one16)

    def stage_body(st, _):
      sbase = e0 + st * STAGE_E
      pltpu.sync_copy(src_hbm.at[pl.ds(sbase, STAGE_E)], src_stage)
      pltpu.sync_copy(dst_hbm.at[pl.ds(sbase, STAGE_E)], dst_stage)

      set_window(0, sv0, dv0)
      start_gather(sv0, rows0, semA)

      def pair_body(j, _):
        k0 = 2 * j
        set_window(k0 + 1, sv1, dv1)
        start_gather(sv1, rows1, semB)
        consume(dv0, rows0, semA)
        set_window(k0 + 2, sv0, dv0)
        start_gather(sv0, rows0, semA)
        consume(dv1, rows1, semB)
        return 0

      lax.fori_loop(0, (SCH - 1) // 2, pair_body, 0)
      consume(dv0, rows0, semA)
      return 0

    lax.fori_loop(0, N_STAGES, stage_body, 0)
    plsc.subcore_barrier()

    pltpu.sync_copy(
        accum.at[pl.ds(row0, ROWS_PER_TILE)],
        out_hbm.at[c, pl.ds(row0, ROWS_PER_TILE)],
    )
    if with_deg:
      pltpu.sync_copy(deg_local, deg_hbm.at[wid])

    @pl.when(s == NS - 1)
    def _copy_tail():
      pltpu.sync_copy(
          accum.at[pl.ds(TAIL_ROW0, TAIL_ROWS)],
          out_hbm.at[c, pl.ds(TAIL_ROW0, TAIL_ROWS)],
      )

  out_type = [jax.ShapeDtypeStruct((NC, N_NODES, D), jnp.float32)]
  if with_deg:
    out_type.append(jax.ShapeDtypeStruct((NW, 1, DEG_PAD), jnp.float32))
  scratch = [
      pltpu.VMEM_SHARED((N_NODES, D), jnp.float32),
      pltpu.VMEM((STAGE_E,), jnp.int32),
      pltpu.VMEM((STAGE_E,), jnp.int32),
      pltpu.VMEM((CHUNK,), jnp.int32),
      pltpu.VMEM((CHUNK,), jnp.int32),
      pltpu.VMEM((CHUNK,), jnp.int32),
      pltpu.VMEM((CHUNK,), jnp.int32),
      pltpu.VMEM((CHUNK, D), jnp.float32),
      pltpu.VMEM((CHUNK, D), jnp.float32),
  ]
  if with_deg:
    scratch.append(pltpu.VMEM((1, DEG_PAD), jnp.float32))
  scratch += [
      pltpu.VMEM((ZROWS, D), jnp.float32),
      pltpu.SemaphoreType.DMA,
      pltpu.SemaphoreType.DMA,
  ]

  return pl.kernel(
      body,
      out_type=tuple(out_type),
      mesh=_MESH,
      scratch_types=tuple(scratch),
      name="sage_agg_deg" if with_deg else "sage_agg",
      compiler_params=pltpu.CompilerParams(needs_layout_passes=False),
  )


_agg_with_deg = _make_agg(True)
_agg_plain = _make_agg(False)


def _mlp_body(x_ref, s1a_ref, s1b_ref, degp_ref, ws1_ref, wn1_ref,
              b1_ref, ws2_ref, wn2_ref, b2_ref, z2_ref, self2_ref, inv_ref):
  deg = jnp.sum(degp_ref[:, 0, :], axis=0)[:, None]
  inv = 1.0 / jnp.maximum(deg, 1.0)
  hn = (s1a_ref[...] + s1b_ref[...]) * inv
  h = x_ref[...] @ ws1_ref[...] + hn @ wn1_ref[...] + b1_ref[...]
  h = jnp.maximum(h, 0.0)
  z2_ref[...] = h @ wn2_ref[...]
  self2_ref[...] = h @ ws2_ref[...] + b2_ref[...]
  inv_ref[...] = jnp.broadcast_to(inv, (BLK, DEG_W))


def _row_spec(cols):
  return pl.BlockSpec((BLK, cols), lambda i: (i, 0))


def _full_spec(r, c):
  return pl.BlockSpec((r, c), lambda i: (0, 0))


_mlp = pl.pallas_call(
    _mlp_body,
    grid=(NBLK,),
    in_specs=[
        _row_spec(D), _row_spec(D), _row_spec(D),
        pl.BlockSpec((NW, 1, BLK), lambda i: (0, 0, i)),
        _full_spec(D, HID), _full_spec(D, HID), _full_spec(1, HID),
        _full_spec(HID, D), _full_spec(HID, D), _full_spec(1, D),
    ],
    out_specs=[_row_spec(D), _row_spec(D), _row_spec(DEG_W)],
    out_shape=[
        jax.ShapeDtypeStruct((N_NODES, D), jnp.float32),
        jax.ShapeDtypeStruct((N_NODES, D), jnp.float32),
        jax.ShapeDtypeStruct((N_NODES, DEG_W), jnp.float32),
    ],
)


def _fin_body(self2_ref, s2a_ref, s2b_ref, inv_ref, out_ref):
  out_ref[...] = (
      self2_ref[...]
      + (s2a_ref[...] + s2b_ref[...]) * inv_ref[...][:, :1]
  )


_fin = pl.pallas_call(
    _fin_body,
    grid=(NBLK,),
    in_specs=[_row_spec(D), _row_spec(D), _row_spec(D), _row_spec(DEG_W)],
    out_specs=_row_spec(D),
    out_shape=jax.ShapeDtypeStruct((N_NODES, D), jnp.float32),
)


def kernel(features, edge_index, W_self1, W_neigh1, b1, W_self2, W_neigh2,
           b2):
  src = edge_index[0].astype(jnp.int32)
  dst = edge_index[1].astype(jnp.int32)

  s1p, degp = _agg_with_deg(features, src, dst)
  z2, self2, inv = _mlp(
      features, s1p[0], s1p[1], degp,
      W_self1, W_neigh1, b1.reshape(1, HID),
      W_self2, W_neigh2, b2.reshape(1, D),
  )
  (s2p,) = _agg_plain(z2, src, dst)
  return _fin(self2, s2p[0], s2p[1], inv)


# speedup vs baseline: 13.3647x; 1.0140x over previous
import functools

import jax
import jax.numpy as jnp
from jax import lax
from jax.experimental import pallas as pl
from jax.experimental.pallas import tpu as pltpu
from jax.experimental.pallas import tpu_sc as plsc

N_NODES = 10000
N_EDGES = 320000
D = 128
HID = 256
DEG_W = 16

NC = 2
NS = 16
NW = NC * NS
E_PER_W = N_EDGES // NW
CHUNK = 80
STAGE_E = 2000
N_STAGES = E_PER_W // STAGE_E
SCH = STAGE_E // CHUNK
ROWS_PER_TILE = 624
TAIL_ROW0 = NS * ROWS_PER_TILE
TAIL_ROWS = N_NODES - TAIL_ROW0
ZROWS = 48
BLK = 1024
NBLK = -(-N_NODES // BLK)
DEG_PAD = NBLK * BLK

_MESH = plsc.VectorSubcoreMesh(
    core_axis_name="c", subcore_axis_name="s", num_cores=NC, num_subcores=NS
)


def _fill_rows(ref, nrows, ncols, value):
  vec = jnp.full((16,), value, jnp.float32)

  def body(r, _):
    for k in range(ncols // 16):
      ref[r, pl.ds(k * 16, 16)] = vec
    return 0

  lax.fori_loop(0, nrows, body, 0)


def _make_agg(with_deg):
  def body(x_hbm, src_hbm, dst_hbm, *rest):
    if with_deg:
      (out_hbm, deg_hbm, accum, src_stage, dst_stage, sv0, sv1, dv0, dv1,
       rows0, rows1, deg_local, zero_v, semA, semB) = rest
    else:
      (out_hbm, accum, src_stage, dst_stage, sv0, sv1, dv0, dv1,
       rows0, rows1, zero_v, semA, semB) = rest
    c = lax.axis_index("c")
    s = lax.axis_index("s")
    wid = (c * NS + s).astype(jnp.int32)
    row0 = s * ROWS_PER_TILE
    e0 = wid * E_PER_W

    _fill_rows(zero_v, ZROWS, D, 0.0)
    if with_deg:
      zvec = jnp.zeros((16,), jnp.float32)

      def zdl(i, _):
        deg_local[0, pl.ds(i * 16, 16)] = zvec
        return 0

      lax.fori_loop(0, DEG_PAD // 16, zdl, 0)

    def zloop(i, _):
      pltpu.sync_copy(zero_v, accum.at[pl.ds(row0 + i * ZROWS, ZROWS)])
      return 0

    lax.fori_loop(0, ROWS_PER_TILE // ZROWS, zloop, 0)

    @pl.when(s == NS - 1)
    def _zero_tail():
      pltpu.sync_copy(
          zero_v.at[pl.ds(0, TAIL_ROWS)], accum.at[pl.ds(TAIL_ROW0, TAIL_ROWS)]
      )

    plsc.subcore_barrier()

    def set_window(k, sv, dv):
      for t in range(CHUNK // 16):
        off = k * CHUNK + t * 16
        sv[pl.ds(t * 16, 16)] = src_stage[pl.ds(off, 16)]
        dv[pl.ds(t * 16, 16)] = dst_stage[pl.ds(off, 16)]

    def start_gather(sv, rows, sem):
      pltpu.async_copy(x_hbm.at[sv], rows, sem)

    def wait_gather(rows, sem):
      pltpu.make_async_copy(x_hbm.at[pl.ds(0, CHUNK)], rows, sem).wait()

    def consume(dv, rows, sem):
      if with_deg:
        one16 = jnp.ones((16,), jnp.float32)
        zrow = jnp.zeros((16,), jnp.int32)
        for kk in range(CHUNK // 16):
          dvec = dv[pl.ds(kk * 16, 16)]
          plsc.addupdate_scatter(deg_local, [zrow, dvec], one16)
      wait_gather(rows, sem)
      pltpu.sync_copy(rows, accum.at[dv], add=True)

    def stage_body(st, _):
      sbase = e0 + st * STAGE_E
      pltpu.sync_copy(src_hbm.at[pl.ds(sbase, STAGE_E)], src_stage)
      pltpu.sync_copy(dst_hbm.at[pl.ds(sbase, STAGE_E)], dst_stage)

      set_window(0, sv0, dv0)
      start_gather(sv0, rows0, semA)

      def pair_body(j, _):
        k0 = 2 * j
        set_window(k0 + 1, sv1, dv1)
        start_gather(sv1, rows1, semB)
        consume(dv0, rows0, semA)
        set_window(k0 + 2, sv0, dv0)
        start_gather(sv0, rows0, semA)
        consume(dv1, rows1, semB)
        return 0

      lax.fori_loop(0, (SCH - 1) // 2, pair_body, 0)
      consume(dv0, rows0, semA)
      return 0

    lax.fori_loop(0, N_STAGES, stage_body, 0)
    plsc.subcore_barrier()

    pltpu.sync_copy(
        accum.at[pl.ds(row0, ROWS_PER_TILE)],
        out_hbm.at[c, pl.ds(row0, ROWS_PER_TILE)],
    )
    if with_deg:
      pltpu.sync_copy(deg_local, deg_hbm.at[wid])

    @pl.when(s == NS - 1)
    def _copy_tail():
      pltpu.sync_copy(
          accum.at[pl.ds(TAIL_ROW0, TAIL_ROWS)],
          out_hbm.at[c, pl.ds(TAIL_ROW0, TAIL_ROWS)],
      )

  out_type = [jax.ShapeDtypeStruct((NC, N_NODES, D), jnp.float32)]
  if with_deg:
    out_type.append(jax.ShapeDtypeStruct((NW, 1, DEG_PAD), jnp.float32))
  scratch = [
      pltpu.VMEM_SHARED((N_NODES, D), jnp.float32),
      pltpu.VMEM((STAGE_E,), jnp.int32),
      pltpu.VMEM((STAGE_E,), jnp.int32),
      pltpu.VMEM((CHUNK,), jnp.int32),
      pltpu.VMEM((CHUNK,), jnp.int32),
      pltpu.VMEM((CHUNK,), jnp.int32),
      pltpu.VMEM((CHUNK,), jnp.int32),
      pltpu.VMEM((CHUNK, D), jnp.float32),
      pltpu.VMEM((CHUNK, D), jnp.float32),
  ]
  if with_deg:
    scratch.append(pltpu.VMEM((1, DEG_PAD), jnp.float32))
  scratch += [
      pltpu.VMEM((ZROWS, D), jnp.float32),
      pltpu.SemaphoreType.DMA,
      pltpu.SemaphoreType.DMA,
  ]

  return pl.kernel(
      body,
      out_type=tuple(out_type),
      mesh=_MESH,
      scratch_types=tuple(scratch),
      name="sage_agg_deg" if with_deg else "sage_agg",
      compiler_params=pltpu.CompilerParams(needs_layout_passes=False),
  )


_agg_with_deg = _make_agg(True)
_agg_plain = _make_agg(False)


def _mlp_body(x_ref, s1a_ref, s1b_ref, degp_ref, ws1_ref, wn1_ref,
              b1_ref, ws2_ref, wn2_ref, b2_ref, z2_ref, self2_ref, inv_ref):
  deg = jnp.sum(degp_ref[:, 0, :], axis=0)[:, None]
  inv = 1.0 / jnp.maximum(deg, 1.0)
  hn = (s1a_ref[...] + s1b_ref[...]) * inv
  h = x_ref[...] @ ws1_ref[...] + hn @ wn1_ref[...] + b1_ref[...]
  h = jnp.maximum(h, 0.0)
  z2_ref[...] = h @ wn2_ref[...]
  self2_ref[...] = h @ ws2_ref[...] + b2_ref[...]
  inv_ref[...] = jnp.broadcast_to(inv, (BLK, DEG_W))


def _row_spec(cols):
  return pl.BlockSpec((BLK, cols), lambda i: (i, 0))


def _full_spec(r, c):
  return pl.BlockSpec((r, c), lambda i: (0, 0))


_mlp = pl.pallas_call(
    _mlp_body,
    grid=(NBLK,),
    in_specs=[
        _row_spec(D), _row_spec(D), _row_spec(D),
        pl.BlockSpec((NW, 1, BLK), lambda i: (0, 0, i)),
        _full_spec(D, HID), _full_spec(D, HID), _full_spec(1, HID),
        _full_spec(HID, D), _full_spec(HID, D), _full_spec(1, D),
    ],
    out_specs=[_row_spec(D), _row_spec(D), _row_spec(DEG_W)],
    out_shape=[
        jax.ShapeDtypeStruct((N_NODES, D), jnp.float32),
        jax.ShapeDtypeStruct((N_NODES, D), jnp.float32),
        jax.ShapeDtypeStruct((N_NODES, DEG_W), jnp.float32),
    ],
)


def _fin_body(self2_ref, s2a_ref, s2b_ref, inv_ref, out_ref):
  out_ref[...] = (
      self2_ref[...]
      + (s2a_ref[...] + s2b_ref[...]) * inv_ref[...][:, :1]
  )


_fin = pl.pallas_call(
    _fin_body,
    grid=(NBLK,),
    in_specs=[_row_spec(D), _row_spec(D), _row_spec(D), _row_spec(DEG_W)],
    out_specs=_row_spec(D),
    out_shape=jax.ShapeDtypeStruct((N_NODES, D), jnp.float32),
)


def kernel(features, edge_index, W_self1, W_neigh1, b1, W_self2, W_neigh2,
           b2):
  src = edge_index[0].astype(jnp.int32)
  dst = edge_index[1].astype(jnp.int32)

  s1p, degp = _agg_with_deg(features, src, dst)
  z2, self2, inv = _mlp(
      features, s1p[0], s1p[1], degp,
      W_self1, W_neigh1, b1.reshape(1, HID),
      W_self2, W_neigh2, b2.reshape(1, D),
  )
  (s2p,) = _agg_plain(z2, src, dst)
  return _fin(self2, s2p[0], s2p[1], inv)


# speedup vs baseline: 15.1744x; 1.1354x over previous
import functools

import jax
import jax.numpy as jnp
from jax import lax
from jax.experimental import pallas as pl
from jax.experimental.pallas import tpu as pltpu
from jax.experimental.pallas import tpu_sc as plsc

N_NODES = 10000
N_EDGES = 320000
D = 128
HID = 256
DEG_W = 16

NC = 2
NS = 16
NW = NC * NS
E_PER_W = N_EDGES // NW
CHUNK = 80
STAGE_E = 2000
N_STAGES = E_PER_W // STAGE_E
SCH = STAGE_E // CHUNK
ROWS_PER_TILE = 624
TAIL_ROW0 = NS * ROWS_PER_TILE
TAIL_ROWS = N_NODES - TAIL_ROW0
ZROWS = 16
BLK = 1024
NBLK = -(-N_NODES // BLK)
DEG_PAD = NBLK * BLK

_MESH = plsc.VectorSubcoreMesh(
    core_axis_name="c", subcore_axis_name="s", num_cores=NC, num_subcores=NS
)


def _fill_rows(ref, nrows, ncols, value):
  vec = jnp.full((16,), value, jnp.float32)

  def body(r, _):
    for k in range(ncols // 16):
      ref[r, pl.ds(k * 16, 16)] = vec
    return 0

  lax.fori_loop(0, nrows, body, 0)


def _make_agg(with_deg):
  def body(x_hbm, src_hbm, dst_hbm, *rest):
    if with_deg:
      (out_hbm, deg_hbm, accum, src_stage, dst_stage, sv0, sv1, sv2,
       dv0, dv1, dv2, rows0, rows1, rows2, deg_local, zero_v,
       gsem0, gsem1, gsem2, ssem0, ssem1, ssem2) = rest
    else:
      (out_hbm, accum, src_stage, dst_stage, sv0, sv1, sv2,
       dv0, dv1, dv2, rows0, rows1, rows2, zero_v,
       gsem0, gsem1, gsem2, ssem0, ssem1, ssem2) = rest
    c = lax.axis_index("c")
    s = lax.axis_index("s")
    wid = (c * NS + s).astype(jnp.int32)
    row0 = s * ROWS_PER_TILE
    e0 = wid * E_PER_W

    _fill_rows(zero_v, ZROWS, D, 0.0)
    if with_deg:
      zvec = jnp.zeros((16,), jnp.float32)

      def zdl(i, _):
        deg_local[0, pl.ds(i * 16, 16)] = zvec
        return 0

      lax.fori_loop(0, DEG_PAD // 16, zdl, 0)

    def zloop(i, _):
      pltpu.sync_copy(zero_v, accum.at[pl.ds(row0 + i * ZROWS, ZROWS)])
      return 0

    lax.fori_loop(0, ROWS_PER_TILE // ZROWS, zloop, 0)

    @pl.when(s == NS - 1)
    def _zero_tail():
      pltpu.sync_copy(
          zero_v.at[pl.ds(0, TAIL_ROWS)], accum.at[pl.ds(TAIL_ROW0, TAIL_ROWS)]
      )

    plsc.subcore_barrier()

    svs = (sv0, sv1, sv2)
    dvs = (dv0, dv1, dv2)
    rows = (rows0, rows1, rows2)
    gsems = (gsem0, gsem1, gsem2)
    ssems = (ssem0, ssem1, ssem2)

    def set_window(k, b):
      for t in range(CHUNK // 16):
        off = k * CHUNK + t * 16
        svs[b][pl.ds(t * 16, 16)] = src_stage[pl.ds(off, 16)]
        dvs[b][pl.ds(t * 16, 16)] = dst_stage[pl.ds(off, 16)]

    def start_gather(k, b):
      set_window(k, b)
      pltpu.async_copy(x_hbm.at[svs[b]], rows[b], gsems[b])

    def wait_gather(b):
      pltpu.make_async_copy(
          x_hbm.at[pl.ds(0, CHUNK)], rows[b], gsems[b]
      ).wait()

    def wait_scatter(b):
      pltpu.make_async_copy(rows[b], accum.at[dvs[b]], ssems[b]).wait()

    def consume(b):
      if with_deg:
        one16 = jnp.ones((16,), jnp.float32)
        zrow = jnp.zeros((16,), jnp.int32)
        for kk in range(CHUNK // 16):
          dvec = dvs[b][pl.ds(kk * 16, 16)]
          plsc.addupdate_scatter(deg_local, [zrow, dvec], one16)
      wait_gather(b)
      pltpu.async_copy(rows[b], accum.at[dvs[b]], ssems[b], add=True)

    def full_body(k, b, first):
      if not first:
        wait_scatter(b)
      start_gather(k, b)
      consume((b + 1) % 3)

    def stage_body(st, _):
      sbase = e0 + st * STAGE_E
      pltpu.sync_copy(src_hbm.at[pl.ds(sbase, STAGE_E)], src_stage)
      pltpu.sync_copy(dst_hbm.at[pl.ds(sbase, STAGE_E)], dst_stage)

      start_gather(0, 0)
      start_gather(1, 1)
      full_body(2, 2, True)

      def trip_body(t, _):
        kb = 3 + 3 * t
        full_body(kb, 0, False)
        full_body(kb + 1, 1, False)
        full_body(kb + 2, 2, False)
        return 0

      lax.fori_loop(0, (SCH - 4) // 3, trip_body, 0)
      full_body(SCH - 1, (SCH - 1) % 3, False)
      consume((SCH - 2) % 3)
      consume((SCH - 1) % 3)
      for b in range(3):
        wait_scatter(b)
      return 0

    lax.fori_loop(0, N_STAGES, stage_body, 0)
    plsc.subcore_barrier()

    pltpu.sync_copy(
        accum.at[pl.ds(row0, ROWS_PER_TILE)],
        out_hbm.at[c, pl.ds(row0, ROWS_PER_TILE)],
    )
    if with_deg:
      pltpu.sync_copy(deg_local, deg_hbm.at[wid])

    @pl.when(s == NS - 1)
    def _copy_tail():
      pltpu.sync_copy(
          accum.at[pl.ds(TAIL_ROW0, TAIL_ROWS)],
          out_hbm.at[c, pl.ds(TAIL_ROW0, TAIL_ROWS)],
      )

  out_type = [jax.ShapeDtypeStruct((NC, N_NODES, D), jnp.float32)]
  if with_deg:
    out_type.append(jax.ShapeDtypeStruct((NW, 1, DEG_PAD), jnp.float32))
  scratch = [
      pltpu.VMEM_SHARED((N_NODES, D), jnp.float32),
      pltpu.VMEM((STAGE_E,), jnp.int32),
      pltpu.VMEM((STAGE_E,), jnp.int32),
  ]
  scratch += [pltpu.VMEM((CHUNK,), jnp.int32)] * 6
  scratch += [pltpu.VMEM((CHUNK, D), jnp.float32)] * 3
  if with_deg:
    scratch.append(pltpu.VMEM((1, DEG_PAD), jnp.float32))
  scratch.append(pltpu.VMEM((ZROWS, D), jnp.float32))
  scratch += [pltpu.SemaphoreType.DMA] * 6

  return pl.kernel(
      body,
      out_type=tuple(out_type),
      mesh=_MESH,
      scratch_types=tuple(scratch),
      name="sage_agg_deg" if with_deg else "sage_agg",
      compiler_params=pltpu.CompilerParams(needs_layout_passes=False),
  )


_agg_with_deg = _make_agg(True)
_agg_plain = _make_agg(False)


def _mlp_body(x_ref, s1a_ref, s1b_ref, degp_ref, ws1_ref, wn1_ref,
              b1_ref, ws2_ref, wn2_ref, b2_ref, z2_ref, self2_ref, inv_ref):
  deg = jnp.sum(degp_ref[:, 0, :], axis=0)[:, None]
  inv = 1.0 / jnp.maximum(deg, 1.0)
  hn = (s1a_ref[...] + s1b_ref[...]) * inv
  h = x_ref[...] @ ws1_ref[...] + hn @ wn1_ref[...] + b1_ref[...]
  h = jnp.maximum(h, 0.0)
  z2_ref[...] = h @ wn2_ref[...]
  self2_ref[...] = h @ ws2_ref[...] + b2_ref[...]
  inv_ref[...] = jnp.broadcast_to(inv, (BLK, DEG_W))


def _row_spec(cols):
  return pl.BlockSpec((BLK, cols), lambda i: (i, 0))


def _full_spec(r, c):
  return pl.BlockSpec((r, c), lambda i: (0, 0))


_mlp = pl.pallas_call(
    _mlp_body,
    grid=(NBLK,),
    in_specs=[
        _row_spec(D), _row_spec(D), _row_spec(D),
        pl.BlockSpec((NW, 1, BLK), lambda i: (0, 0, i)),
        _full_spec(D, HID), _full_spec(D, HID), _full_spec(1, HID),
        _full_spec(HID, D), _full_spec(HID, D), _full_spec(1, D),
    ],
    out_specs=[_row_spec(D), _row_spec(D), _row_spec(DEG_W)],
    out_shape=[
        jax.ShapeDtypeStruct((N_NODES, D), jnp.float32),
        jax.ShapeDtypeStruct((N_NODES, D), jnp.float32),
        jax.ShapeDtypeStruct((N_NODES, DEG_W), jnp.float32),
    ],
)


def _fin_body(self2_ref, s2a_ref, s2b_ref, inv_ref, out_ref):
  out_ref[...] = (
      self2_ref[...]
      + (s2a_ref[...] + s2b_ref[...]) * inv_ref[...][:, :1]
  )


_fin = pl.pallas_call(
    _fin_body,
    grid=(NBLK,),
    in_specs=[_row_spec(D), _row_spec(D), _row_spec(D), _row_spec(DEG_W)],
    out_specs=_row_spec(D),
    out_shape=jax.ShapeDtypeStruct((N_NODES, D), jnp.float32),
)


def kernel(features, edge_index, W_self1, W_neigh1, b1, W_self2, W_neigh2,
           b2):
  src = edge_index[0].astype(jnp.int32)
  dst = edge_index[1].astype(jnp.int32)

  s1p, degp = _agg_with_deg(features, src, dst)
  z2, self2, inv = _mlp(
      features, s1p[0], s1p[1], degp,
      W_self1, W_neigh1, b1.reshape(1, HID),
      W_self2, W_neigh2, b2.reshape(1, D),
  )
  (s2p,) = _agg_plain(z2, src, dst)
  return _fin(self2, s2p[0], s2p[1], inv)


# speedup vs baseline: 15.1896x; 1.0010x over previous
import functools

import jax
import jax.numpy as jnp
from jax import lax
from jax.experimental import pallas as pl
from jax.experimental.pallas import tpu as pltpu
from jax.experimental.pallas import tpu_sc as plsc

N_NODES = 10000
N_EDGES = 320000
D = 128
HID = 256
DEG_W = 16

NC = 2
NS = 16
NW = NC * NS
E_PER_W = N_EDGES // NW
CHUNK = 80
STAGE_E = 2000
N_STAGES = E_PER_W // STAGE_E
SCH = STAGE_E // CHUNK
ROWS_PER_TILE = 624
TAIL_ROW0 = NS * ROWS_PER_TILE
TAIL_ROWS = N_NODES - TAIL_ROW0
ZROWS = 16
BLK = 1024
NBLK = -(-N_NODES // BLK)
DEG_PAD = NBLK * BLK

_MESH = plsc.VectorSubcoreMesh(
    core_axis_name="c", subcore_axis_name="s", num_cores=NC, num_subcores=NS
)


def _fill_rows(ref, nrows, ncols, value):
  vec = jnp.full((16,), value, jnp.float32)

  def body(r, _):
    for k in range(ncols // 16):
      ref[r, pl.ds(k * 16, 16)] = vec
    return 0

  lax.fori_loop(0, nrows, body, 0)


def _make_agg(with_deg):
  def body(x_hbm, src_hbm, dst_hbm, *rest):
    if with_deg:
      (out_hbm, deg_hbm, accum, src_stage, dst_stage,
       dv0, dv1, dv2, rows0, rows1, rows2, deg_local, zero_v,
       gsem0, gsem1, gsem2, ssem0, ssem1, ssem2) = rest
    else:
      (out_hbm, accum, src_stage, dst_stage,
       dv0, dv1, dv2, rows0, rows1, rows2, zero_v,
       gsem0, gsem1, gsem2, ssem0, ssem1, ssem2) = rest
    c = lax.axis_index("c")
    s = lax.axis_index("s")
    wid = (c * NS + s).astype(jnp.int32)
    row0 = s * ROWS_PER_TILE
    e0 = wid * E_PER_W

    _fill_rows(zero_v, ZROWS, D, 0.0)
    if with_deg:
      zvec = jnp.zeros((16,), jnp.float32)

      def zdl(i, _):
        deg_local[0, pl.ds(i * 16, 16)] = zvec
        return 0

      lax.fori_loop(0, DEG_PAD // 16, zdl, 0)

    def zloop(i, _):
      pltpu.sync_copy(zero_v, accum.at[pl.ds(row0 + i * ZROWS, ZROWS)])
      return 0

    lax.fori_loop(0, ROWS_PER_TILE // ZROWS, zloop, 0)

    @pl.when(s == NS - 1)
    def _zero_tail():
      pltpu.sync_copy(
          zero_v.at[pl.ds(0, TAIL_ROWS)], accum.at[pl.ds(TAIL_ROW0, TAIL_ROWS)]
      )

    plsc.subcore_barrier()

    dvs = (dv0, dv1, dv2)
    rows = (rows0, rows1, rows2)
    gsems = (gsem0, gsem1, gsem2)
    ssems = (ssem0, ssem1, ssem2)

    def set_window(k, b):
      for t in range(CHUNK // 16):
        off = k * CHUNK + t * 16
        dvs[b][pl.ds(t * 16, 16)] = dst_stage[pl.ds(off, 16)]

    def start_gather(k, b):
      set_window(k, b)
      pltpu.async_copy(
          x_hbm.at[src_stage.at[pl.ds(k * CHUNK, CHUNK)]], rows[b], gsems[b]
      )

    def wait_gather(b):
      pltpu.make_async_copy(
          x_hbm.at[pl.ds(0, CHUNK)], rows[b], gsems[b]
      ).wait()

    def wait_scatter(b):
      pltpu.make_async_copy(rows[b], accum.at[dvs[b]], ssems[b]).wait()

    def consume(b):
      if with_deg:
        one16 = jnp.ones((16,), jnp.float32)
        zrow = jnp.zeros((16,), jnp.int32)
        for kk in range(CHUNK // 16):
          dvec = dvs[b][pl.ds(kk * 16, 16)]
          plsc.addupdate_scatter(deg_local, [zrow, dvec], one16)
      wait_gather(b)
      pltpu.async_copy(rows[b], accum.at[dvs[b]], ssems[b], add=True)

    def full_body(k, b, first):
      if not first:
        wait_scatter(b)
      start_gather(k, b)
      consume((b + 1) % 3)

    def stage_body(st, _):
      sbase = e0 + st * STAGE_E
      pltpu.sync_copy(src_hbm.at[pl.ds(sbase, STAGE_E)], src_stage)
      pltpu.sync_copy(dst_hbm.at[pl.ds(sbase, STAGE_E)], dst_stage)

      start_gather(0, 0)
      start_gather(1, 1)
      full_body(2, 2, True)

      def trip_body(t, _):
        kb = 3 + 3 * t
        full_body(kb, 0, False)
        full_body(kb + 1, 1, False)
        full_body(kb + 2, 2, False)
        return 0

      lax.fori_loop(0, (SCH - 4) // 3, trip_body, 0)
      full_body(SCH - 1, (SCH - 1) % 3, False)
      consume((SCH - 2) % 3)
      consume((SCH - 1) % 3)
      for b in range(3):
        wait_scatter(b)
      return 0

    lax.fori_loop(0, N_STAGES, stage_body, 0)
    plsc.subcore_barrier()

    pltpu.sync_copy(
        accum.at[pl.ds(row0, ROWS_PER_TILE)],
        out_hbm.at[c, pl.ds(row0, ROWS_PER_TILE)],
    )
    if with_deg:
      pltpu.sync_copy(deg_local, deg_hbm.at[wid])

    @pl.when(s == NS - 1)
    def _copy_tail():
      pltpu.sync_copy(
          accum.at[pl.ds(TAIL_ROW0, TAIL_ROWS)],
          out_hbm.at[c, pl.ds(TAIL_ROW0, TAIL_ROWS)],
      )

  out_type = [jax.ShapeDtypeStruct((NC, N_NODES, D), jnp.float32)]
  if with_deg:
    out_type.append(jax.ShapeDtypeStruct((NW, 1, DEG_PAD), jnp.float32))
  scratch = [
      pltpu.VMEM_SHARED((N_NODES, D), jnp.float32),
      pltpu.VMEM((STAGE_E,), jnp.int32),
      pltpu.VMEM((STAGE_E,), jnp.int32),
  ]
  scratch += [pltpu.VMEM((CHUNK,), jnp.int32)] * 3
  scratch += [pltpu.VMEM((CHUNK, D), jnp.float32)] * 3
  if with_deg:
    scratch.append(pltpu.VMEM((1, DEG_PAD), jnp.float32))
  scratch.append(pltpu.VMEM((ZROWS, D), jnp.float32))
  scratch += [pltpu.SemaphoreType.DMA] * 6

  return pl.kernel(
      body,
      out_type=tuple(out_type),
      mesh=_MESH,
      scratch_types=tuple(scratch),
      name="sage_agg_deg" if with_deg else "sage_agg",
      compiler_params=pltpu.CompilerParams(needs_layout_passes=False),
  )


_agg_with_deg = _make_agg(True)
_agg_plain = _make_agg(False)


def _mlp_body(x_ref, s1a_ref, s1b_ref, degp_ref, ws1_ref, wn1_ref,
              b1_ref, ws2_ref, wn2_ref, b2_ref, z2_ref, self2_ref, inv_ref):
  deg = jnp.sum(degp_ref[:, 0, :], axis=0)[:, None]
  inv = 1.0 / jnp.maximum(deg, 1.0)
  hn = (s1a_ref[...] + s1b_ref[...]) * inv
  h = x_ref[...] @ ws1_ref[...] + hn @ wn1_ref[...] + b1_ref[...]
  h = jnp.maximum(h, 0.0)
  z2_ref[...] = h @ wn2_ref[...]
  self2_ref[...] = h @ ws2_ref[...] + b2_ref[...]
  inv_ref[...] = jnp.broadcast_to(inv, (BLK, DEG_W))


def _row_spec(cols):
  return pl.BlockSpec((BLK, cols), lambda i: (i, 0))


def _full_spec(r, c):
  return pl.BlockSpec((r, c), lambda i: (0, 0))


_mlp = pl.pallas_call(
    _mlp_body,
    grid=(NBLK,),
    in_specs=[
        _row_spec(D), _row_spec(D), _row_spec(D),
        pl.BlockSpec((NW, 1, BLK), lambda i: (0, 0, i)),
        _full_spec(D, HID), _full_spec(D, HID), _full_spec(1, HID),
        _full_spec(HID, D), _full_spec(HID, D), _full_spec(1, D),
    ],
    out_specs=[_row_spec(D), _row_spec(D), _row_spec(DEG_W)],
    out_shape=[
        jax.ShapeDtypeStruct((N_NODES, D), jnp.float32),
        jax.ShapeDtypeStruct((N_NODES, D), jnp.float32),
        jax.ShapeDtypeStruct((N_NODES, DEG_W), jnp.float32),
    ],
)


def _fin_body(self2_ref, s2a_ref, s2b_ref, inv_ref, out_ref):
  out_ref[...] = (
      self2_ref[...]
      + (s2a_ref[...] + s2b_ref[...]) * inv_ref[...][:, :1]
  )


_fin = pl.pallas_call(
    _fin_body,
    grid=(NBLK,),
    in_specs=[_row_spec(D), _row_spec(D), _row_spec(D), _row_spec(DEG_W)],
    out_specs=_row_spec(D),
    out_shape=jax.ShapeDtypeStruct((N_NODES, D), jnp.float32),
)


def kernel(features, edge_index, W_self1, W_neigh1, b1, W_self2, W_neigh2,
           b2):
  src = edge_index[0].astype(jnp.int32)
  dst = edge_index[1].astype(jnp.int32)

  s1p, degp = _agg_with_deg(features, src, dst)
  z2, self2, inv = _mlp(
      features, s1p[0], s1p[1], degp,
      W_self1, W_neigh1, b1.reshape(1, HID),
      W_self2, W_neigh2, b2.reshape(1, D),
  )
  (s2p,) = _agg_plain(z2, src, dst)
  return _fin(self2, s2p[0], s2p[1], inv)


# speedup vs baseline: 16.0471x; 1.0565x over previous
import functools

import jax
import jax.numpy as jnp
from jax import lax
from jax.experimental import pallas as pl
from jax.experimental.pallas import tpu as pltpu
from jax.experimental.pallas import tpu_sc as plsc

N_NODES = 10000
N_EDGES = 320000
D = 128
HID = 256
DEG_W = 16

NC = 2
NS = 16
NW = NC * NS
E_PER_W = N_EDGES // NW
CHUNK = 80
STAGE_E = 2000
N_STAGES = E_PER_W // STAGE_E
SCH = STAGE_E // CHUNK
ROWS_PER_TILE = 624
TAIL_ROW0 = NS * ROWS_PER_TILE
TAIL_ROWS = N_NODES - TAIL_ROW0
ZROWS = 8
BLK = 1024
NBLK = -(-N_NODES // BLK)
DEG_PAD = NBLK * BLK

_MESH = plsc.VectorSubcoreMesh(
    core_axis_name="c", subcore_axis_name="s", num_cores=NC, num_subcores=NS
)


def _fill_rows(ref, nrows, ncols, value):
  vec = jnp.full((16,), value, jnp.float32)

  def body(r, _):
    for k in range(ncols // 16):
      ref[r, pl.ds(k * 16, 16)] = vec
    return 0

  lax.fori_loop(0, nrows, body, 0)


def _make_agg(with_deg):
  def body(x_hbm, src_hbm, dst_hbm, *rest):
    if with_deg:
      (out_hbm, deg_hbm, accum, src_stage0, dst_stage0, src_stage1,
       dst_stage1, dv0, dv1, dv2, rows0, rows1, rows2, deg_local, zero_v,
       gsem0, gsem1, gsem2, ssem0, ssem1, ssem2, isem0, isem1) = rest
    else:
      (out_hbm, accum, src_stage0, dst_stage0, src_stage1,
       dst_stage1, dv0, dv1, dv2, rows0, rows1, rows2, zero_v,
       gsem0, gsem1, gsem2, ssem0, ssem1, ssem2, isem0, isem1) = rest
    c = lax.axis_index("c")
    s = lax.axis_index("s")
    wid = (c * NS + s).astype(jnp.int32)
    row0 = s * ROWS_PER_TILE
    e0 = wid * E_PER_W

    sstages = (src_stage0, src_stage1)
    dstages = (dst_stage0, dst_stage1)
    isems = (isem0, isem1)

    def start_stage_load(st, sb):
      sbase = e0 + st * STAGE_E
      pltpu.async_copy(src_hbm.at[pl.ds(sbase, STAGE_E)], sstages[sb],
                       isems[sb])
      pltpu.async_copy(dst_hbm.at[pl.ds(sbase, STAGE_E)], dstages[sb],
                       isems[sb])

    def wait_stage_load(sb):
      pltpu.make_async_copy(src_hbm.at[pl.ds(0, STAGE_E)], sstages[sb],
                            isems[sb]).wait()
      pltpu.make_async_copy(dst_hbm.at[pl.ds(0, STAGE_E)], dstages[sb],
                            isems[sb]).wait()

    start_stage_load(0, 0)

    _fill_rows(zero_v, ZROWS, D, 0.0)
    if with_deg:
      zvec = jnp.zeros((16,), jnp.float32)

      def zdl(i, _):
        deg_local[0, pl.ds(i * 16, 16)] = zvec
        return 0

      lax.fori_loop(0, DEG_PAD // 16, zdl, 0)

    def zloop(i, _):
      pltpu.async_copy(zero_v, accum.at[pl.ds(row0 + i * ZROWS, ZROWS)],
                       gsem0)
      return 0

    lax.fori_loop(0, ROWS_PER_TILE // ZROWS, zloop, 0)

    @pl.when(s == NS - 1)
    def _zero_tail():
      for h in range(TAIL_ROWS // ZROWS):
        pltpu.async_copy(
            zero_v, accum.at[pl.ds(TAIL_ROW0 + h * ZROWS, ZROWS)], gsem0
        )

    def zdrain(i, _):
      pltpu.make_async_copy(
          zero_v, accum.at[pl.ds(row0, ZROWS)], gsem0
      ).wait()
      return 0

    lax.fori_loop(0, ROWS_PER_TILE // ZROWS, zdrain, 0)

    @pl.when(s == NS - 1)
    def _zdrain_tail():
      for h in range(TAIL_ROWS // ZROWS):
        pltpu.make_async_copy(
            zero_v, accum.at[pl.ds(TAIL_ROW0 + h * ZROWS, ZROWS)], gsem0
        ).wait()

    plsc.subcore_barrier()

    dvs = (dv0, dv1, dv2)
    rows = (rows0, rows1, rows2)
    gsems = (gsem0, gsem1, gsem2)
    ssems = (ssem0, ssem1, ssem2)

    def make_set_window(dst_stage):
      def set_window(k, b):
        for t in range(CHUNK // 16):
          off = k * CHUNK + t * 16
          dvs[b][pl.ds(t * 16, 16)] = dst_stage[pl.ds(off, 16)]
      return set_window

    def make_start_gather(src_stage, set_window):
      def start_gather(k, b):
        set_window(k, b)
        pltpu.async_copy(
            x_hbm.at[src_stage.at[pl.ds(k * CHUNK, CHUNK)]], rows[b],
            gsems[b]
        )
      return start_gather

    def wait_gather(b):
      pltpu.make_async_copy(
          x_hbm.at[pl.ds(0, CHUNK)], rows[b], gsems[b]
      ).wait()

    def wait_scatter(b):
      pltpu.make_async_copy(rows[b], accum.at[dvs[b]], ssems[b]).wait()

    def consume(b):
      if with_deg:
        one16 = jnp.ones((16,), jnp.float32)
        zrow = jnp.zeros((16,), jnp.int32)
        for kk in range(CHUNK // 16):
          dvec = dvs[b][pl.ds(kk * 16, 16)]
          plsc.addupdate_scatter(deg_local, [zrow, dvec], one16)
      wait_gather(b)
      pltpu.async_copy(rows[b], accum.at[dvs[b]], ssems[b], add=True)

    def run_stage(sb, st):
      set_window = make_set_window(dstages[sb])
      start_gather = make_start_gather(sstages[sb], set_window)

      def full_body(k, b, first):
        if not first:
          wait_scatter(b)
        start_gather(k, b)
        consume((b + 1) % 3)

      wait_stage_load(sb)

      @pl.when(st < N_STAGES - 1)
      def _prefetch():
        start_stage_load(st + 1, 1 - sb)

      start_gather(0, 0)
      start_gather(1, 1)
      full_body(2, 2, True)

      def trip_body(t, _):
        kb = 3 + 3 * t
        full_body(kb, 0, False)
        full_body(kb + 1, 1, False)
        full_body(kb + 2, 2, False)
        return 0

      lax.fori_loop(0, (SCH - 4) // 3, trip_body, 0)
      full_body(SCH - 1, (SCH - 1) % 3, False)
      consume((SCH - 2) % 3)
      consume((SCH - 1) % 3)
      for b in range(3):
        wait_scatter(b)

    def stage_pair(p, _):
      run_stage(0, 2 * p)
      run_stage(1, 2 * p + 1)
      return 0

    lax.fori_loop(0, N_STAGES // 2, stage_pair, 0)
    if N_STAGES % 2:
      run_stage(0, N_STAGES - 1)
    plsc.subcore_barrier()

    pltpu.sync_copy(
        accum.at[pl.ds(row0, ROWS_PER_TILE)],
        out_hbm.at[c, pl.ds(row0, ROWS_PER_TILE)],
    )
    if with_deg:
      pltpu.sync_copy(deg_local, deg_hbm.at[wid])

    @pl.when(s == NS - 1)
    def _copy_tail():
      pltpu.sync_copy(
          accum.at[pl.ds(TAIL_ROW0, TAIL_ROWS)],
          out_hbm.at[c, pl.ds(TAIL_ROW0, TAIL_ROWS)],
      )

  out_type = [jax.ShapeDtypeStruct((NC, N_NODES, D), jnp.float32)]
  if with_deg:
    out_type.append(jax.ShapeDtypeStruct((NW, 1, DEG_PAD), jnp.float32))
  scratch = [
      pltpu.VMEM_SHARED((N_NODES, D), jnp.float32),
  ]
  scratch += [pltpu.VMEM((STAGE_E,), jnp.int32)] * 4
  scratch += [pltpu.VMEM((CHUNK,), jnp.int32)] * 3
  scratch += [pltpu.VMEM((CHUNK, D), jnp.float32)] * 3
  if with_deg:
    scratch.append(pltpu.VMEM((1, DEG_PAD), jnp.float32))
  scratch.append(pltpu.VMEM((ZROWS, D), jnp.float32))
  scratch += [pltpu.SemaphoreType.DMA] * 8

  return pl.kernel(
      body,
      out_type=tuple(out_type),
      mesh=_MESH,
      scratch_types=tuple(scratch),
      name="sage_agg_deg" if with_deg else "sage_agg",
      compiler_params=pltpu.CompilerParams(needs_layout_passes=False),
  )


_agg_with_deg = _make_agg(True)
_agg_plain = _make_agg(False)


def _mlp_body(x_ref, s1a_ref, s1b_ref, degp_ref, ws1_ref, wn1_ref,
              b1_ref, ws2_ref, wn2_ref, b2_ref, z2_ref, self2_ref, inv_ref):
  deg = jnp.sum(degp_ref[:, 0, :], axis=0)[:, None]
  inv = 1.0 / jnp.maximum(deg, 1.0)
  hn = (s1a_ref[...] + s1b_ref[...]) * inv
  h = x_ref[...] @ ws1_ref[...] + hn @ wn1_ref[...] + b1_ref[...]
  h = jnp.maximum(h, 0.0)
  z2_ref[...] = h @ wn2_ref[...]
  self2_ref[...] = h @ ws2_ref[...] + b2_ref[...]
  inv_ref[...] = jnp.broadcast_to(inv, (BLK, DEG_W))


def _row_spec(cols):
  return pl.BlockSpec((BLK, cols), lambda i: (i, 0))


def _full_spec(r, c):
  return pl.BlockSpec((r, c), lambda i: (0, 0))


_mlp = pl.pallas_call(
    _mlp_body,
    grid=(NBLK,),
    in_specs=[
        _row_spec(D), _row_spec(D), _row_spec(D),
        pl.BlockSpec((NW, 1, BLK), lambda i: (0, 0, i)),
        _full_spec(D, HID), _full_spec(D, HID), _full_spec(1, HID),
        _full_spec(HID, D), _full_spec(HID, D), _full_spec(1, D),
    ],
    out_specs=[_row_spec(D), _row_spec(D), _row_spec(DEG_W)],
    out_shape=[
        jax.ShapeDtypeStruct((N_NODES, D), jnp.float32),
        jax.ShapeDtypeStruct((N_NODES, D), jnp.float32),
        jax.ShapeDtypeStruct((N_NODES, DEG_W), jnp.float32),
    ],
)


def _fin_body(self2_ref, s2a_ref, s2b_ref, inv_ref, out_ref):
  out_ref[...] = (
      self2_ref[...]
      + (s2a_ref[...] + s2b_ref[...]) * inv_ref[...][:, :1]
  )


_fin = pl.pallas_call(
    _fin_body,
    grid=(NBLK,),
    in_specs=[_row_spec(D), _row_spec(D), _row_spec(D), _row_spec(DEG_W)],
    out_specs=_row_spec(D),
    out_shape=jax.ShapeDtypeStruct((N_NODES, D), jnp.float32),
)


def kernel(features, edge_index, W_self1, W_neigh1, b1, W_self2, W_neigh2,
           b2):
  src = edge_index[0].astype(jnp.int32)
  dst = edge_index[1].astype(jnp.int32)

  s1p, degp = _agg_with_deg(features, src, dst)
  z2, self2, inv = _mlp(
      features, s1p[0], s1p[1], degp,
      W_self1, W_neigh1, b1.reshape(1, HID),
      W_self2, W_neigh2, b2.reshape(1, D),
  )
  (s2p,) = _agg_plain(z2, src, dst)
  return _fin(self2, s2p[0], s2p[1], inv)


# speedup vs baseline: 16.1175x; 1.0044x over previous
import functools

import jax
import jax.numpy as jnp
from jax import lax
from jax.experimental import pallas as pl
from jax.experimental.pallas import tpu as pltpu
from jax.experimental.pallas import tpu_sc as plsc

N_NODES = 10000
N_EDGES = 320000
D = 128
HID = 256
DEG_W = 16

NC = 2
NS = 16
NW = NC * NS
E_PER_W = N_EDGES // NW
CHUNK = 80
STAGE_E = 2000
N_STAGES = E_PER_W // STAGE_E
SCH = STAGE_E // CHUNK
ROWS_PER_TILE = 624
TAIL_ROW0 = NS * ROWS_PER_TILE
TAIL_ROWS = N_NODES - TAIL_ROW0
ZROWS = 8
BLK = 1024
NBLK = -(-N_NODES // BLK)
DEG_PAD = NBLK * BLK

_MESH = plsc.VectorSubcoreMesh(
    core_axis_name="c", subcore_axis_name="s", num_cores=NC, num_subcores=NS
)


def _fill_rows(ref, nrows, ncols, value):
  vec = jnp.full((16,), value, jnp.float32)

  def body(r, _):
    for k in range(ncols // 16):
      ref[r, pl.ds(k * 16, 16)] = vec
    return 0

  lax.fori_loop(0, nrows, body, 0)


def _make_agg(with_deg):
  def body(x_hbm, src_hbm, dst_hbm, *rest):
    if with_deg:
      (out_hbm, deg_hbm, accum, src_stage0, dst_stage0, src_stage1,
       dst_stage1, dv0, dv1, dv2, rows0, rows1, rows2, deg_local, zero_v,
       gsem0, gsem1, gsem2, ssem0, ssem1, ssem2, isem0, isem1) = rest
    else:
      (out_hbm, accum, src_stage0, dst_stage0, src_stage1,
       dst_stage1, dv0, dv1, dv2, rows0, rows1, rows2, zero_v,
       gsem0, gsem1, gsem2, ssem0, ssem1, ssem2, isem0, isem1) = rest
    c = lax.axis_index("c")
    s = lax.axis_index("s")
    wid = (c * NS + s).astype(jnp.int32)
    row0 = s * ROWS_PER_TILE
    e0 = wid * E_PER_W

    sstages = (src_stage0, src_stage1)
    dstages = (dst_stage0, dst_stage1)
    isems = (isem0, isem1)

    def start_stage_load(st, sb):
      sbase = e0 + st * STAGE_E
      pltpu.async_copy(src_hbm.at[pl.ds(sbase, STAGE_E)], sstages[sb],
                       isems[sb])
      pltpu.async_copy(dst_hbm.at[pl.ds(sbase, STAGE_E)], dstages[sb],
                       isems[sb])

    def wait_stage_load(sb):
      pltpu.make_async_copy(src_hbm.at[pl.ds(0, STAGE_E)], sstages[sb],
                            isems[sb]).wait()
      pltpu.make_async_copy(dst_hbm.at[pl.ds(0, STAGE_E)], dstages[sb],
                            isems[sb]).wait()

    start_stage_load(0, 0)

    _fill_rows(zero_v, ZROWS, D, 0.0)
    if with_deg:
      zvec = jnp.zeros((16,), jnp.float32)

      def zdl(i, _):
        deg_local[0, pl.ds(i * 16, 16)] = zvec
        return 0

      lax.fori_loop(0, DEG_PAD // 16, zdl, 0)

    def zloop(i, _):
      pltpu.async_copy(zero_v, accum.at[pl.ds(row0 + i * ZROWS, ZROWS)],
                       gsem0)
      return 0

    lax.fori_loop(0, ROWS_PER_TILE // ZROWS, zloop, 0)

    @pl.when(s == NS - 1)
    def _zero_tail():
      for h in range(TAIL_ROWS // ZROWS):
        pltpu.async_copy(
            zero_v, accum.at[pl.ds(TAIL_ROW0 + h * ZROWS, ZROWS)], gsem0
        )

    def zdrain(i, _):
      pltpu.make_async_copy(
          zero_v, accum.at[pl.ds(row0, ZROWS)], gsem0
      ).wait()
      return 0

    lax.fori_loop(0, ROWS_PER_TILE // ZROWS, zdrain, 0)

    @pl.when(s == NS - 1)
    def _zdrain_tail():
      for h in range(TAIL_ROWS // ZROWS):
        pltpu.make_async_copy(
            zero_v, accum.at[pl.ds(TAIL_ROW0 + h * ZROWS, ZROWS)], gsem0
        ).wait()

    plsc.subcore_barrier()

    dvs = (dv0, dv1, dv2)
    rows = (rows0, rows1, rows2)
    gsems = (gsem0, gsem1, gsem2)
    ssems = (ssem0, ssem1, ssem2)

    def make_set_window(dst_stage):
      def set_window(k, b):
        for t in range(CHUNK // 16):
          off = k * CHUNK + t * 16
          dvs[b][pl.ds(t * 16, 16)] = dst_stage[pl.ds(off, 16)]
      return set_window

    def make_start_gather(src_stage, set_window):
      def start_gather(k, b):
        set_window(k, b)
        pltpu.async_copy(
            x_hbm.at[src_stage.at[pl.ds(k * CHUNK, CHUNK)]], rows[b],
            gsems[b]
        )
      return start_gather

    def wait_gather(b):
      pltpu.make_async_copy(
          x_hbm.at[pl.ds(0, CHUNK)], rows[b], gsems[b]
      ).wait()

    def wait_scatter(b):
      pltpu.make_async_copy(rows[b], accum.at[dvs[b]], ssems[b]).wait()

    def consume(b):
      if with_deg:
        one16 = jnp.ones((16,), jnp.float32)
        zrow = jnp.zeros((16,), jnp.int32)
        for kk in range(CHUNK // 16):
          dvec = dvs[b][pl.ds(kk * 16, 16)]
          plsc.addupdate_scatter(deg_local, [zrow, dvec], one16)
      wait_gather(b)
      pltpu.async_copy(rows[b], accum.at[dvs[b]], ssems[b], add=True)

    def run_stage(sb, st):
      set_window = make_set_window(dstages[sb])
      start_gather = make_start_gather(sstages[sb], set_window)

      def full_body(k, b, first):
        if not first:
          wait_scatter(b)
        start_gather(k, b)
        consume((b + 1) % 3)

      wait_stage_load(sb)

      @pl.when(st < N_STAGES - 1)
      def _prefetch():
        start_stage_load(st + 1, 1 - sb)

      start_gather(0, 0)
      start_gather(1, 1)
      full_body(2, 2, True)

      def trip_body(t, _):
        kb = 3 + 3 * t
        full_body(kb, 0, False)
        full_body(kb + 1, 1, False)
        full_body(kb + 2, 2, False)
        return 0

      lax.fori_loop(0, (SCH - 4) // 3, trip_body, 0)
      full_body(SCH - 1, (SCH - 1) % 3, False)
      consume((SCH - 2) % 3)
      consume((SCH - 1) % 3)
      for b in range(3):
        wait_scatter(b)

    def stage_pair(p, _):
      run_stage(0, 2 * p)
      run_stage(1, 2 * p + 1)
      return 0

    lax.fori_loop(0, N_STAGES // 2, stage_pair, 0)
    if N_STAGES % 2:
      run_stage(0, N_STAGES - 1)
    plsc.subcore_barrier()

    pltpu.sync_copy(
        accum.at[pl.ds(row0, ROWS_PER_TILE)],
        out_hbm.at[c, pl.ds(row0, ROWS_PER_TILE)],
    )
    if with_deg:
      pltpu.sync_copy(deg_local, deg_hbm.at[wid])

    @pl.when(s == NS - 1)
    def _copy_tail():
      pltpu.sync_copy(
          accum.at[pl.ds(TAIL_ROW0, TAIL_ROWS)],
          out_hbm.at[c, pl.ds(TAIL_ROW0, TAIL_ROWS)],
      )

  out_type = [jax.ShapeDtypeStruct((NC, N_NODES, D), jnp.float32)]
  if with_deg:
    out_type.append(jax.ShapeDtypeStruct((NW, 1, DEG_PAD), jnp.float32))
  scratch = [
      pltpu.VMEM_SHARED((N_NODES, D), jnp.float32),
  ]
  scratch += [pltpu.VMEM((STAGE_E,), jnp.int32)] * 4
  scratch += [pltpu.VMEM((CHUNK,), jnp.int32)] * 3
  scratch += [pltpu.VMEM((CHUNK, D), jnp.float32)] * 3
  if with_deg:
    scratch.append(pltpu.VMEM((1, DEG_PAD), jnp.float32))
  scratch.append(pltpu.VMEM((ZROWS, D), jnp.float32))
  scratch += [pltpu.SemaphoreType.DMA] * 8

  return pl.kernel(
      body,
      out_type=tuple(out_type),
      mesh=_MESH,
      scratch_types=tuple(scratch),
      name="sage_agg_deg" if with_deg else "sage_agg",
      compiler_params=pltpu.CompilerParams(needs_layout_passes=False),
  )


_agg_with_deg = _make_agg(True)
_agg_plain = _make_agg(False)


def _hself_body(x_ref, ws1_ref, b1_ref, hs_ref):
  hs_ref[...] = x_ref[...] @ ws1_ref[...] + b1_ref[...]


_hself = pl.pallas_call(
    _hself_body,
    grid=(NBLK,),
    in_specs=[
        pl.BlockSpec((BLK, D), lambda i: (i, 0)),
        pl.BlockSpec((D, HID), lambda i: (0, 0)),
        pl.BlockSpec((1, HID), lambda i: (0, 0)),
    ],
    out_specs=pl.BlockSpec((BLK, HID), lambda i: (i, 0)),
    out_shape=jax.ShapeDtypeStruct((N_NODES, HID), jnp.float32),
)


def _mlp_body(hs_ref, s1a_ref, s1b_ref, degp_ref, wn1_ref,
              ws2_ref, wn2_ref, b2_ref, z2_ref, self2_ref, inv_ref):
  deg = jnp.sum(degp_ref[:, 0, :], axis=0)[:, None]
  inv = 1.0 / jnp.maximum(deg, 1.0)
  hn = (s1a_ref[...] + s1b_ref[...]) * inv
  h = jnp.maximum(hs_ref[...] + hn @ wn1_ref[...], 0.0)
  z2_ref[...] = h @ wn2_ref[...]
  self2_ref[...] = h @ ws2_ref[...] + b2_ref[...]
  inv_ref[...] = jnp.broadcast_to(inv, (BLK, DEG_W))


def _row_spec(cols):
  return pl.BlockSpec((BLK, cols), lambda i: (i, 0))


def _full_spec(r, c):
  return pl.BlockSpec((r, c), lambda i: (0, 0))


_mlp = pl.pallas_call(
    _mlp_body,
    grid=(NBLK,),
    in_specs=[
        pl.BlockSpec((BLK, HID), lambda i: (i, 0)),
        _row_spec(D), _row_spec(D),
        pl.BlockSpec((NW, 1, BLK), lambda i: (0, 0, i)),
        _full_spec(D, HID),
        _full_spec(HID, D), _full_spec(HID, D), _full_spec(1, D),
    ],
    out_specs=[_row_spec(D), _row_spec(D), _row_spec(DEG_W)],
    out_shape=[
        jax.ShapeDtypeStruct((N_NODES, D), jnp.float32),
        jax.ShapeDtypeStruct((N_NODES, D), jnp.float32),
        jax.ShapeDtypeStruct((N_NODES, DEG_W), jnp.float32),
    ],
)


def _fin_body(self2_ref, s2a_ref, s2b_ref, inv_ref, out_ref):
  out_ref[...] = (
      self2_ref[...]
      + (s2a_ref[...] + s2b_ref[...]) * inv_ref[...][:, :1]
  )


_fin = pl.pallas_call(
    _fin_body,
    grid=(NBLK,),
    in_specs=[_row_spec(D), _row_spec(D), _row_spec(D), _row_spec(DEG_W)],
    out_specs=_row_spec(D),
    out_shape=jax.ShapeDtypeStruct((N_NODES, D), jnp.float32),
)


def kernel(features, edge_index, W_self1, W_neigh1, b1, W_self2, W_neigh2,
           b2):
  src = edge_index[0].astype(jnp.int32)
  dst = edge_index[1].astype(jnp.int32)

  hs = _hself(features, W_self1, b1.reshape(1, HID))
  s1p, degp = _agg_with_deg(features, src, dst)
  z2, self2, inv = _mlp(
      hs, s1p[0], s1p[1], degp,
      W_neigh1, W_self2, W_neigh2, b2.reshape(1, D),
  )
  (s2p,) = _agg_plain(z2, src, dst)
  return _fin(self2, s2p[0], s2p[1], inv)
